# Initial kernel scaffold; baseline (speedup 1.0000x reference)
import jax
import jax.numpy as jnp
from jax.experimental import pallas as pl


def kernel(x, ei, W1, b1, W2, b2):
    raise NotImplementedError("write your pallas kernel here")


# speedup vs baseline: 10.4991x; 10.4991x over previous
import functools

import jax
import jax.numpy as jnp
from jax import lax
from jax.experimental import pallas as pl
from jax.experimental.pallas import tpu as pltpu
from jax.experimental.pallas import tpu_sc as plsc

N_PAD = 10240
CHUNK = 64
CHUNK_DEG = 128
NC, NS = 2, 16
NW = NC * NS
ROWS_PER_TILE = N_PAD // NS
BLK = 512


def _sc_mesh():
    return plsc.VectorSubcoreMesh(
        core_axis_name="c", subcore_axis_name="s", num_cores=NC, num_subcores=NS
    )


def _make_deg_kernel(n_chunks):
    @functools.partial(
        pl.kernel,
        mesh=_sc_mesh(),
        out_type=jax.ShapeDtypeStruct((NC * N_PAD,), jnp.float32),
        scratch_types=[
            pltpu.VMEM_SHARED((N_PAD,), jnp.float32),
            pltpu.VMEM((n_chunks, CHUNK_DEG), jnp.int32),
            pltpu.VMEM((CHUNK_DEG,), jnp.float32),
            pltpu.VMEM((ROWS_PER_TILE,), jnp.float32),
        ],
    )
    def deg_kernel(dst_hbm, out_hbm, deg_sp, didx, ones_v, zeros_v):
        cid = lax.axis_index("c")
        sid = lax.axis_index("s")
        wid = cid * NS + sid
        for i in range(CHUNK_DEG // 16):
            ones_v[pl.ds(i * 16, 16)] = jnp.ones((16,), jnp.float32)

        def zinit(i, carry):
            zeros_v[pl.ds(i * 16, 16)] = jnp.zeros((16,), jnp.float32)
            return carry

        lax.fori_loop(0, ROWS_PER_TILE // 16, zinit, 0)
        row0 = sid * ROWS_PER_TILE
        pltpu.sync_copy(zeros_v, deg_sp.at[pl.ds(row0, ROWS_PER_TILE)])
        pltpu.sync_copy(dst_hbm.at[pl.ds(wid * n_chunks, n_chunks)], didx)
        plsc.subcore_barrier()

        def body(c, carry):
            pltpu.sync_copy(ones_v, deg_sp.at[didx.at[c]], add=True)
            return carry

        lax.fori_loop(0, n_chunks, body, 0)
        plsc.subcore_barrier()
        pltpu.sync_copy(deg_sp.at[pl.ds(row0, ROWS_PER_TILE)], zeros_v)
        pltpu.sync_copy(zeros_v, out_hbm.at[pl.ds(cid * N_PAD + row0, ROWS_PER_TILE)])

    return deg_kernel


def _make_hop_kernel(n_chunks, d):
    @functools.partial(
        pl.kernel,
        mesh=_sc_mesh(),
        out_type=jax.ShapeDtypeStruct((NC, N_PAD, d), jnp.float32),
        scratch_types=[
            pltpu.VMEM_SHARED((N_PAD, d), jnp.float32),
            pltpu.VMEM((4, CHUNK), jnp.int32),
            pltpu.VMEM((4, CHUNK), jnp.int32),
            pltpu.VMEM((4, CHUNK, d), jnp.float32),
            [pltpu.SemaphoreType.DMA] * 4,
            [pltpu.SemaphoreType.DMA] * 4,
        ],
    )
    def hop_kernel(g_hbm, src_hbm, dst_hbm, out_hbm,
                   acc_sp, sidx, didx, rows, sgs, sis):
        cid = lax.axis_index("c")
        sid = lax.axis_index("s")
        wid = cid * NS + sid
        row0 = sid * ROWS_PER_TILE
        base_e = wid * n_chunks * CHUNK
        n_stage = ROWS_PER_TILE // CHUNK

        def idx_load(c, q):
            off = base_e + c * CHUNK
            pltpu.async_copy(src_hbm.at[pl.ds(off, CHUNK)], sidx.at[q], sis[q])
            pltpu.async_copy(dst_hbm.at[pl.ds(off, CHUNK)], didx.at[q], sis[q])

        def idx_wait(q):
            pltpu.make_async_copy(
                src_hbm.at[pl.ds(0, CHUNK)], sidx.at[q], sis[q]).wait()
            pltpu.make_async_copy(
                dst_hbm.at[pl.ds(0, CHUNK)], didx.at[q], sis[q]).wait()

        def gather_start(q):
            pltpu.async_copy(g_hbm.at[sidx.at[q]], rows.at[q], sgs[q])

        def gather_wait(q):
            pltpu.make_async_copy(g_hbm.at[sidx.at[q]], rows.at[q], sgs[q]).wait()

        cps = [None, None]
        cps[0] = pltpu.async_copy(g_hbm.at[pl.ds(row0, CHUNK)], rows.at[0], sgs[0])
        for i in range(n_stage):
            b = i % 2
            cps[b].wait()
            if i + 1 < n_stage:
                cps[1 - b] = pltpu.async_copy(
                    g_hbm.at[pl.ds(row0 + (i + 1) * CHUNK, CHUNK)],
                    rows.at[1 - b], sgs[1 - b])
            pltpu.sync_copy(rows.at[b], acc_sp.at[pl.ds(row0 + i * CHUNK, CHUNK)])
        plsc.subcore_barrier()

        for q in range(4):
            idx_load(q, q)
        for q in range(3):
            idx_wait(q)
            gather_start(q)

        def body(c4, carry):
            for j in range(4):
                cur = c4 * 4 + j
                qn = (j + 3) % 4
                gather_wait(j)
                pltpu.sync_copy(rows.at[j], acc_sp.at[didx.at[j]], add=True)

                @pl.when(cur + 4 < n_chunks)
                def _():
                    idx_load(cur + 4, j)

                @pl.when(cur + 3 < n_chunks)
                def _():
                    idx_wait(qn)
                    gather_start(qn)

            return carry

        lax.fori_loop(0, n_chunks // 4, body, 0)
        plsc.subcore_barrier()
        wps = [None, None]
        for i in range(n_stage):
            b = i % 2
            if wps[b] is not None:
                wps[b].wait()
            pltpu.sync_copy(acc_sp.at[pl.ds(row0 + i * CHUNK, CHUNK)], rows.at[b])
            wps[b] = pltpu.async_copy(
                rows.at[b], out_hbm.at[cid, pl.ds(row0 + i * CHUNK, CHUNK)], sgs[b])
        for w in wps:
            if w is not None:
                w.wait()

    return hop_kernel


def _mm1(x_pad, w1t, b1r, deg0, deg1):
    d = x_pad.shape[1]
    h = w1t.shape[1]

    def body(x_ref, w_ref, b_ref, d0_ref, d1_ref, g_ref, ds_ref, dv_ref):
        deg = d0_ref[...] + d1_ref[...] + 1.0
        ds = lax.rsqrt(deg)
        hh = jnp.dot(x_ref[...], w_ref[...],
                     preferred_element_type=jnp.float32,
                     precision=lax.Precision.HIGHEST)
        hh = jnp.maximum(hh + b_ref[...], 0.0)
        g_ref[...] = hh * ds
        ds_ref[...] = ds
        dv_ref[...] = 1.0 / deg

    return pl.pallas_call(
        body,
        grid=(N_PAD // BLK,),
        in_specs=[
            pl.BlockSpec((BLK, d), lambda i: (i, 0)),
            pl.BlockSpec((d, h), lambda i: (0, 0)),
            pl.BlockSpec((1, h), lambda i: (0, 0)),
            pl.BlockSpec((BLK, 1), lambda i: (i, 0)),
            pl.BlockSpec((BLK, 1), lambda i: (i, 0)),
        ],
        out_specs=[
            pl.BlockSpec((BLK, h), lambda i: (i, 0)),
            pl.BlockSpec((BLK, 1), lambda i: (i, 0)),
            pl.BlockSpec((BLK, 1), lambda i: (i, 0)),
        ],
        out_shape=[
            jax.ShapeDtypeStruct((N_PAD, h), jnp.float32),
            jax.ShapeDtypeStruct((N_PAD, 1), jnp.float32),
            jax.ShapeDtypeStruct((N_PAD, 1), jnp.float32),
        ],
    )(x_pad, w1t, b1r, deg0, deg1)


def _combine(p, g, dnv):
    d = g.shape[1]

    def body(p0_ref, p1_ref, g_ref, dv_ref, o_ref):
        o_ref[...] = (p0_ref[0] + p1_ref[0] - g_ref[...]) * dv_ref[...]

    return pl.pallas_call(
        body,
        grid=(N_PAD // BLK,),
        in_specs=[
            pl.BlockSpec((1, BLK, d), lambda i: (0, i, 0)),
            pl.BlockSpec((1, BLK, d), lambda i: (1, i, 0)),
            pl.BlockSpec((BLK, d), lambda i: (i, 0)),
            pl.BlockSpec((BLK, 1), lambda i: (i, 0)),
        ],
        out_specs=pl.BlockSpec((BLK, d), lambda i: (i, 0)),
        out_shape=jax.ShapeDtypeStruct((N_PAD, d), jnp.float32),
    )(p, p, g, dnv)


def _mm2(q, g, dsq, w2t, b2r):
    h = q.shape[2]
    o = w2t.shape[1]

    def body(q0_ref, q1_ref, g_ref, ds_ref, w_ref, b_ref, o_ref):
        t = (q0_ref[0] + q1_ref[0] - g_ref[...]) * ds_ref[...]
        o_ref[...] = jnp.dot(t, w_ref[...],
                             preferred_element_type=jnp.float32,
                             precision=lax.Precision.HIGHEST) + b_ref[...]

    return pl.pallas_call(
        body,
        grid=(N_PAD // BLK,),
        in_specs=[
            pl.BlockSpec((1, BLK, h), lambda i: (0, i, 0)),
            pl.BlockSpec((1, BLK, h), lambda i: (1, i, 0)),
            pl.BlockSpec((BLK, h), lambda i: (i, 0)),
            pl.BlockSpec((BLK, 1), lambda i: (i, 0)),
            pl.BlockSpec((h, o), lambda i: (0, 0)),
            pl.BlockSpec((1, o), lambda i: (0, 0)),
        ],
        out_specs=pl.BlockSpec((BLK, o), lambda i: (i, 0)),
        out_shape=jax.ShapeDtypeStruct((N_PAD, o), jnp.float32),
    )(q, q, g, dsq, w2t, b2r)


def kernel(x, ei, W1, b1, W2, b2):
    n, d = x.shape
    e = ei.shape[1]
    n_chunks = -(-e // (NW * CHUNK))
    n_chunks = -(-n_chunks // 16) * 16
    e_pad = NW * n_chunks * CHUNK
    n_chunks_deg = e_pad // (NW * CHUNK_DEG)

    src = jnp.concatenate(
        [ei[0].astype(jnp.int32), jnp.zeros((e_pad - e,), jnp.int32)]
    )
    dst = jnp.concatenate(
        [ei[1].astype(jnp.int32), jnp.full((e_pad - e,), n, jnp.int32)]
    )
    dst2d = dst.reshape(-1, CHUNK_DEG)

    x_pad = jnp.zeros((N_PAD, d), jnp.float32).at[:n].set(x.astype(jnp.float32))
    w1t = W1.T.astype(jnp.float32)
    w2t = W2.T.astype(jnp.float32)
    b1r = b1.reshape(1, -1).astype(jnp.float32)
    b2r = b2.reshape(1, -1).astype(jnp.float32)

    deg_k = _make_deg_kernel(n_chunks_deg)
    hop_k = _make_hop_kernel(n_chunks, w1t.shape[1])

    deg2 = deg_k(dst2d).reshape(NC, N_PAD)
    deg0 = deg2[0].reshape(N_PAD, 1)
    deg1 = deg2[1].reshape(N_PAD, 1)

    g0, dsq, dnv = _mm1(x_pad, w1t, b1r, deg0, deg1)
    p = hop_k(g0, src, dst)
    g1 = _combine(p, g0, dnv)
    q = hop_k(g1, src, dst)
    out = _mm2(q, g1, dsq, w2t, b2r)
    return out[:n]
